# Initial kernel scaffold; baseline (speedup 1.0000x reference)
import jax
import jax.numpy as jnp
from jax.experimental import pallas as pl


def kernel(tokens, real_table, imag_table, W1, b1, W2, b2):
    raise NotImplementedError("write your pallas kernel here")


# speedup vs baseline: 5.1758x; 5.1758x over previous
import dataclasses
import functools

import jax
import jax.numpy as jnp
import numpy as np
from jax import lax
from jax.experimental import pallas as pl
from jax.experimental.pallas import tpu as pltpu
from jax.experimental.pallas import tpu_sc as plsc

_LANES = 16
_EPS = np.float32(1e-8)
_MAGIC = np.int32(0x5F3759DF)


def _rsqrt_approx(t):
    i = plsc.bitcast(t, jnp.int32)
    i = _MAGIC - lax.shift_right_arithmetic(i, 1)
    y = plsc.bitcast(i, jnp.float32)
    th = t * np.float32(0.5)
    y = y * (np.float32(1.5) - th * y * y)
    y = y * (np.float32(1.5) - th * y * y)
    return y


def _np_zero_phase():
    t = _EPS
    i = np.int32(np.float32(t).view(np.int32))
    i = np.int32(_MAGIC - (i >> np.int32(1)))
    y = np.int32(i).view(np.float32)
    th = np.float32(t * np.float32(0.5))
    y = np.float32(y * (np.float32(1.5) - th * y * y))
    y = np.float32(y * (np.float32(1.5) - th * y * y))
    return np.float32(t * y)


_ZERO_PHASE = _np_zero_phase()


def _pooled_sc(tokens, real_table, imag_table):
    B, L = tokens.shape
    V, D = real_table.shape
    NC, NS = 2, 16
    NW = NC * NS
    BW = B // NW
    NCH = D // _LANES
    mesh = plsc.VectorSubcoreMesh(core_axis_name="c", subcore_axis_name="s")
    cp = pltpu.CompilerParams()
    if "needs_layout_passes" in pltpu.CompilerParams.__dataclass_fields__:
        cp = dataclasses.replace(cp, needs_layout_passes=False)

    @functools.partial(
        pl.kernel,
        mesh=mesh,
        compiler_params=cp,
        out_type=jax.ShapeDtypeStruct((B, D), jnp.float32),
        scratch_types=[
            pltpu.VMEM((BW, L), jnp.int32),
            pltpu.VMEM((L, D), jnp.float32),
            pltpu.VMEM((L, D), jnp.float32),
            pltpu.VMEM((BW, D), jnp.float32),
            pltpu.SemaphoreType.DMA,
            pltpu.SemaphoreType.DMA,
        ],
    )
    def sc_kernel(tok_hbm, re_hbm, im_hbm, out_hbm, tok_v, re_v, im_v,
                  pooled_v, sem_r, sem_i):
        wid = lax.axis_index("s") * NC + lax.axis_index("c")
        base = wid * BW
        pltpu.sync_copy(tok_hbm.at[pl.ds(base, BW)], tok_v)

        zero_v = jnp.zeros((_LANES,), jnp.float32)
        one_v = jnp.full((_LANES,), np.float32(1.0))

        @pl.loop(0, BW)
        def _(s):
            rc = pltpu.async_copy(re_hbm.at[tok_v.at[s]], re_v, sem_r)
            ic = pltpu.async_copy(im_hbm.at[tok_v.at[s]], im_v, sem_i)

            cntv = zero_v
            for c in range(NCH):
                tok = tok_v[s, pl.ds(_LANES * c, _LANES)]
                cntv = cntv + jnp.where(tok != 0, one_v, zero_v)
            cnt = jnp.broadcast_to(jnp.sum(cntv), (_LANES,))

            rc.wait()
            ic.wait()

            def row_body(r, accs):
                nxt = []
                for c in range(NCH):
                    re = re_v[r, pl.ds(_LANES * c, _LANES)]
                    im = im_v[r, pl.ds(_LANES * c, _LANES)]
                    t = re * re + im * im + _EPS
                    nxt.append(accs[c] + t * _rsqrt_approx(t))
                return tuple(nxt)

            accs = lax.fori_loop(0, L, row_body, (zero_v,) * NCH)

            inv = jnp.where(cnt > zero_v, one_v / (cnt + _EPS), zero_v)
            corr = (np.float32(L) - cnt) * _ZERO_PHASE
            for c in range(NCH):
                pooled_v[s, pl.ds(_LANES * c, _LANES)] = (accs[c] - corr) * inv

        pltpu.sync_copy(pooled_v, out_hbm.at[pl.ds(base, BW)])

    return sc_kernel(tokens, real_table, imag_table)


def _mlp_tc(pooled, W1, b1, W2, b2):
    B, D = pooled.shape
    blk = 1024

    def body(x_ref, w1_ref, b1_ref, w2_ref, b2_ref, o_ref):
        h = jnp.dot(x_ref[...], w1_ref[...],
                    preferred_element_type=jnp.float32) + b1_ref[...]
        h = jnp.maximum(h, 0.0)
        o_ref[...] = jnp.dot(h, w2_ref[...],
                             preferred_element_type=jnp.float32) + b2_ref[...]

    return pl.pallas_call(
        body,
        grid=(B // blk,),
        in_specs=[
            pl.BlockSpec((blk, D), lambda i: (i, 0)),
            pl.BlockSpec((D, D), lambda i: (0, 0)),
            pl.BlockSpec((1, D), lambda i: (0, 0)),
            pl.BlockSpec((D, D), lambda i: (0, 0)),
            pl.BlockSpec((1, D), lambda i: (0, 0)),
        ],
        out_specs=pl.BlockSpec((blk, D), lambda i: (i, 0)),
        out_shape=jax.ShapeDtypeStruct((B, D), jnp.float32),
    )(pooled, W1, b1, W2, b2)


def kernel(tokens, real_table, imag_table, W1, b1, W2, b2):
    pooled = _pooled_sc(tokens, real_table, imag_table)
    return _mlp_tc(pooled, W1, b1[None, :], W2, b2[None, :])


# speedup vs baseline: 7.8227x; 1.5114x over previous
import dataclasses
import functools

import jax
import jax.numpy as jnp
import numpy as np
from jax import lax
from jax.experimental import pallas as pl
from jax.experimental.pallas import tpu as pltpu
from jax.experimental.pallas import tpu_sc as plsc

_LANES = 16
_EPS = np.float32(1e-8)
_MAGIC = np.int32(0x5F3759DF)


def _rsqrt_approx(t):
    i = plsc.bitcast(t, jnp.int32)
    i = _MAGIC - lax.shift_right_arithmetic(i, 1)
    y = plsc.bitcast(i, jnp.float32)
    th = t * np.float32(0.5)
    y = y * (np.float32(1.5) - th * y * y)
    y = y * (np.float32(1.5) - th * y * y)
    return y


def _np_zero_phase():
    t = _EPS
    i = np.int32(np.float32(t).view(np.int32))
    i = np.int32(_MAGIC - (i >> np.int32(1)))
    y = np.int32(i).view(np.float32)
    th = np.float32(t * np.float32(0.5))
    y = np.float32(y * (np.float32(1.5) - th * y * y))
    y = np.float32(y * (np.float32(1.5) - th * y * y))
    return np.float32(t * y)


_ZERO_PHASE = _np_zero_phase()


def _pooled_sc(tokens, real_table, imag_table):
    B, L = tokens.shape
    V, D = real_table.shape
    NC, NS = 2, 16
    NW = NC * NS
    BW = B // NW
    NCH = D // _LANES
    mesh = plsc.VectorSubcoreMesh(core_axis_name="c", subcore_axis_name="s")
    cp = pltpu.CompilerParams()
    if "needs_layout_passes" in pltpu.CompilerParams.__dataclass_fields__:
        cp = dataclasses.replace(cp, needs_layout_passes=False)

    @functools.partial(
        pl.kernel,
        mesh=mesh,
        compiler_params=cp,
        out_type=jax.ShapeDtypeStruct((B, D), jnp.float32),
        scratch_types=[
            pltpu.VMEM((BW, L), jnp.int32),
            pltpu.VMEM((2, L, D), jnp.float32),
            pltpu.VMEM((2, L, D), jnp.float32),
            pltpu.VMEM((BW, D), jnp.float32),
            pltpu.SemaphoreType.DMA,
            pltpu.SemaphoreType.DMA,
            pltpu.SemaphoreType.DMA,
            pltpu.SemaphoreType.DMA,
        ],
    )
    def sc_kernel(tok_hbm, re_hbm, im_hbm, out_hbm, tok_v, re_v, im_v,
                  pooled_v, sem_r0, sem_r1, sem_i0, sem_i1):
        wid = lax.axis_index("s") * NC + lax.axis_index("c")
        base = wid * BW
        pltpu.sync_copy(tok_hbm.at[pl.ds(base, BW)], tok_v)

        zero_v = jnp.zeros((_LANES,), jnp.float32)
        one_v = jnp.full((_LANES,), np.float32(1.0))
        sems_r = (sem_r0, sem_r1)
        sems_i = (sem_i0, sem_i1)

        def issue(s, slot):
            pltpu.async_copy(re_hbm.at[tok_v.at[s]], re_v.at[slot],
                             sems_r[slot])
            pltpu.async_copy(im_hbm.at[tok_v.at[s]], im_v.at[slot],
                             sems_i[slot])

        def wait(s, slot):
            pltpu.make_async_copy(re_hbm.at[tok_v.at[s]], re_v.at[slot],
                                  sems_r[slot]).wait()
            pltpu.make_async_copy(im_hbm.at[tok_v.at[s]], im_v.at[slot],
                                  sems_i[slot]).wait()

        def compute(s, slot):
            cntv = zero_v
            for c in range(NCH):
                tok = tok_v[s, pl.ds(_LANES * c, _LANES)]
                cntv = cntv + jnp.where(tok != 0, one_v, zero_v)
            cnt = jnp.broadcast_to(jnp.sum(cntv), (_LANES,))

            def row_body(r, accs):
                nxt = []
                for c in range(NCH):
                    re = re_v[slot, r, pl.ds(_LANES * c, _LANES)]
                    im = im_v[slot, r, pl.ds(_LANES * c, _LANES)]
                    t = re * re + im * im + _EPS
                    nxt.append(accs[c] + t * _rsqrt_approx(t))
                return tuple(nxt)

            accs = lax.fori_loop(0, L, row_body, (zero_v,) * NCH)

            inv = jnp.where(cnt > zero_v, one_v / (cnt + _EPS), zero_v)
            corr = (np.float32(L) - cnt) * _ZERO_PHASE
            for c in range(NCH):
                pooled_v[s, pl.ds(_LANES * c, _LANES)] = (accs[c] - corr) * inv

        issue(0, 0)
        issue(1, 1)

        @pl.loop(0, BW - 2, step=2)
        def _(s0):
            for j in range(2):
                s = s0 + j
                wait(s, j)
                compute(s, j)
                issue(s + 2, j)

        for j in range(2):
            s = BW - 2 + j
            wait(s, j)
            compute(s, j)

        pltpu.sync_copy(pooled_v, out_hbm.at[pl.ds(base, BW)])

    return sc_kernel(tokens, real_table, imag_table)


def _mlp_tc(pooled, W1, b1, W2, b2):
    B, D = pooled.shape
    blk = 1024

    def body(x_ref, w1_ref, b1_ref, w2_ref, b2_ref, o_ref):
        h = jnp.dot(x_ref[...], w1_ref[...],
                    preferred_element_type=jnp.float32) + b1_ref[...]
        h = jnp.maximum(h, 0.0)
        o_ref[...] = jnp.dot(h, w2_ref[...],
                             preferred_element_type=jnp.float32) + b2_ref[...]

    return pl.pallas_call(
        body,
        grid=(B // blk,),
        in_specs=[
            pl.BlockSpec((blk, D), lambda i: (i, 0)),
            pl.BlockSpec((D, D), lambda i: (0, 0)),
            pl.BlockSpec((1, D), lambda i: (0, 0)),
            pl.BlockSpec((D, D), lambda i: (0, 0)),
            pl.BlockSpec((1, D), lambda i: (0, 0)),
        ],
        out_specs=pl.BlockSpec((blk, D), lambda i: (i, 0)),
        out_shape=jax.ShapeDtypeStruct((B, D), jnp.float32),
    )(pooled, W1, b1, W2, b2)


def kernel(tokens, real_table, imag_table, W1, b1, W2, b2):
    pooled = _pooled_sc(tokens, real_table, imag_table)
    return _mlp_tc(pooled, W1, b1[None, :], W2, b2[None, :])


# speedup vs baseline: 10.8698x; 1.3895x over previous
import dataclasses
import functools

import jax
import jax.numpy as jnp
import numpy as np
from jax import lax
from jax.experimental import pallas as pl
from jax.experimental.pallas import tpu as pltpu
from jax.experimental.pallas import tpu_sc as plsc

_LANES = 16
_EPS = np.float32(1e-8)
_MAGIC = np.int32(0x5F3759DF)


def _rsqrt_approx(t):
    i = plsc.bitcast(t, jnp.int32)
    i = _MAGIC - lax.shift_right_arithmetic(i, 1)
    y = plsc.bitcast(i, jnp.float32)
    th = t * np.float32(0.5)
    y = y * (np.float32(1.5) - th * y * y)
    return y


def _pooled_sc(tokens, real_table, imag_table):
    B, L = tokens.shape
    V, D = real_table.shape
    NC, NS = 2, 16
    NW = NC * NS
    BW = B // NW
    NCH = D // _LANES
    mesh = plsc.VectorSubcoreMesh(core_axis_name="c", subcore_axis_name="s")
    cp = pltpu.CompilerParams()
    if "needs_layout_passes" in pltpu.CompilerParams.__dataclass_fields__:
        cp = dataclasses.replace(cp, needs_layout_passes=False)

    @functools.partial(
        pl.kernel,
        mesh=mesh,
        compiler_params=cp,
        out_type=jax.ShapeDtypeStruct((B, D), jnp.float32),
        scratch_types=[
            pltpu.VMEM((BW, L), jnp.int32),
            pltpu.VMEM((2, L, D), jnp.float32),
            pltpu.VMEM((2, L, D), jnp.float32),
            pltpu.VMEM((BW, D), jnp.float32),
            pltpu.SemaphoreType.DMA,
            pltpu.SemaphoreType.DMA,
            pltpu.SemaphoreType.DMA,
            pltpu.SemaphoreType.DMA,
        ],
    )
    def sc_kernel(tok_hbm, re_hbm, im_hbm, out_hbm, tok_v, re_v, im_v,
                  pooled_v, sem_r0, sem_r1, sem_i0, sem_i1):
        wid = lax.axis_index("s") * NC + lax.axis_index("c")
        base = wid * BW
        pltpu.sync_copy(tok_hbm.at[pl.ds(base, BW)], tok_v)

        zero_v = jnp.zeros((_LANES,), jnp.float32)
        one_v = jnp.full((_LANES,), np.float32(1.0))
        sems_r = (sem_r0, sem_r1)
        sems_i = (sem_i0, sem_i1)

        def issue(s, slot):
            pltpu.async_copy(re_hbm.at[tok_v.at[s]], re_v.at[slot],
                             sems_r[slot])
            pltpu.async_copy(im_hbm.at[tok_v.at[s]], im_v.at[slot],
                             sems_i[slot])

        def wait(s, slot):
            pltpu.make_async_copy(re_hbm.at[tok_v.at[s]], re_v.at[slot],
                                  sems_r[slot]).wait()
            pltpu.make_async_copy(im_hbm.at[tok_v.at[s]], im_v.at[slot],
                                  sems_i[slot]).wait()

        def compute(s, slot):
            cntv = zero_v
            for c in range(NCH):
                tok = tok_v[s, pl.ds(_LANES * c, _LANES)]
                cntv = cntv + jnp.where(tok != 0, one_v, zero_v)
            cnt = jnp.broadcast_to(jnp.sum(cntv), (_LANES,))

            def row_body(r, accs):
                nxt = []
                for c in range(NCH):
                    re = re_v[slot, r, pl.ds(_LANES * c, _LANES)]
                    im = im_v[slot, r, pl.ds(_LANES * c, _LANES)]
                    t = re * re + im * im
                    nxt.append(accs[c] + t * _rsqrt_approx(t))
                return tuple(nxt)

            accs = lax.fori_loop(0, L, row_body, (zero_v,) * NCH)

            inv = jnp.where(cnt > zero_v, one_v / (cnt + _EPS), zero_v)
            for c in range(NCH):
                pooled_v[s, pl.ds(_LANES * c, _LANES)] = accs[c] * inv

        issue(0, 0)
        issue(1, 1)

        @pl.loop(0, BW - 2, step=2)
        def _(s0):
            for j in range(2):
                s = s0 + j
                wait(s, j)
                compute(s, j)
                issue(s + 2, j)

        for j in range(2):
            s = BW - 2 + j
            wait(s, j)
            compute(s, j)

        pltpu.sync_copy(pooled_v, out_hbm.at[pl.ds(base, BW)])

    return sc_kernel(tokens, real_table, imag_table)


def _mlp_tc(pooled, W1, b1, W2, b2):
    B, D = pooled.shape
    blk = 1024

    def body(x_ref, w1_ref, b1_ref, w2_ref, b2_ref, o_ref):
        h = jnp.dot(x_ref[...], w1_ref[...],
                    preferred_element_type=jnp.float32) + b1_ref[...]
        h = jnp.maximum(h, 0.0)
        o_ref[...] = jnp.dot(h, w2_ref[...],
                             preferred_element_type=jnp.float32) + b2_ref[...]

    return pl.pallas_call(
        body,
        grid=(B // blk,),
        in_specs=[
            pl.BlockSpec((blk, D), lambda i: (i, 0)),
            pl.BlockSpec((D, D), lambda i: (0, 0)),
            pl.BlockSpec((1, D), lambda i: (0, 0)),
            pl.BlockSpec((D, D), lambda i: (0, 0)),
            pl.BlockSpec((1, D), lambda i: (0, 0)),
        ],
        out_specs=pl.BlockSpec((blk, D), lambda i: (i, 0)),
        out_shape=jax.ShapeDtypeStruct((B, D), jnp.float32),
    )(pooled, W1, b1, W2, b2)


def kernel(tokens, real_table, imag_table, W1, b1, W2, b2):
    pooled = _pooled_sc(tokens, real_table, imag_table)
    return _mlp_tc(pooled, W1, b1[None, :], W2, b2[None, :])
